# baseline (speedup 1.0000x reference)
import functools

import jax
import jax.numpy as jnp
from jax import lax
from jax.experimental import pallas as pl
from jax.experimental.pallas import tpu as pltpu
from jax.experimental.pallas import tpu_sc as plsc

TB = 256
ROWS_B = 512
CHUNK = 32
NBUF = 3


def _argmax_body(x_ref, w1t_ref, b1_ref, w2t_ref, b2_ref, idx_ref):
    h = jnp.maximum(
        jnp.dot(x_ref[...], w1t_ref[...], preferred_element_type=jnp.float32)
        + b1_ref[...], 0.0)
    s = jnp.dot(h, w2t_ref[...], preferred_element_type=jnp.float32) + b2_ref[...]
    m = jnp.max(s, axis=-1, keepdims=True)
    col = lax.broadcasted_iota(jnp.int32, s.shape, 1)
    idx = jnp.min(jnp.where(s == m, col, jnp.int32(2**30)), axis=-1)
    idx_ref[0, 0, :] = idx


def _logsoftmax_body(wt_ref, b_ref, out_ref):
    z = wt_ref[...] + b_ref[...]
    m = jnp.max(z, axis=-1, keepdims=True)
    e = jnp.exp(z - m)
    lse = m + jnp.log(jnp.sum(e, axis=-1, keepdims=True))
    out_ref[...] = z - lse


def _make_gather(n_tokens, d):
    info = plsc.get_sparse_core_info()
    nc, ns = info.num_cores, info.num_subcores
    nw = nc * ns
    b_per_w = n_tokens // nw
    n_chunks = b_per_w // CHUNK
    mesh = plsc.VectorSubcoreMesh(core_axis_name="c", subcore_axis_name="s")

    @functools.partial(
        pl.kernel,
        mesh=mesh,
        out_type=jax.ShapeDtypeStruct((n_tokens, d), jnp.float32),
        scratch_types=[
            pltpu.VMEM((n_chunks, CHUNK), jnp.int32),
            pltpu.VMEM((NBUF, CHUNK, d), jnp.float32),
            pltpu.SemaphoreType.DMA,
            pltpu.SemaphoreType.DMA,
        ],
    )
    def gather_k(table_hbm, idx_hbm, out_hbm, idx_v, rows_v, sem_g, sem_o):
        wid = lax.axis_index("s") * nc + lax.axis_index("c")
        base = wid * b_per_w
        pltpu.sync_copy(idx_hbm.at[wid], idx_v)
        g = [None] * n_chunks
        o = [None] * n_chunks
        for i in range(n_chunks):
            b = i % NBUF
            if i >= NBUF:
                o[i - NBUF].wait()
            g[i] = pltpu.async_copy(table_hbm.at[idx_v.at[i]], rows_v.at[b],
                                    sem_g)
            if i >= 1:
                g[i - 1].wait()
                o[i - 1] = pltpu.async_copy(
                    rows_v.at[(i - 1) % NBUF],
                    out_hbm.at[pl.ds(base + (i - 1) * CHUNK, CHUNK)], sem_o)
        g[n_chunks - 1].wait()
        o[n_chunks - 1] = pltpu.async_copy(
            rows_v.at[(n_chunks - 1) % NBUF],
            out_hbm.at[pl.ds(base + (n_chunks - 1) * CHUNK, CHUNK)], sem_o)
        for i in range(max(0, n_chunks - NBUF), n_chunks):
            o[i].wait()

    return gather_k


def kernel(input, hidden1, W_i2m, b_i2m, W_m2h, b_m2h, W_h2o, b_h2o):
    B, S, I = input.shape
    N = B * S
    H = W_i2m.shape[0]
    C = W_m2h.shape[0]
    O = W_h2o.shape[0]

    x = input.reshape(N, I)
    w1t = W_i2m.T
    w2t = W_m2h.T
    b1 = b_i2m.reshape(1, H)
    b2 = b_m2h.reshape(1, C)

    nb = N // TB
    idx3 = pl.pallas_call(
        _argmax_body,
        grid=(nb,),
        in_specs=[
            pl.BlockSpec((TB, I), lambda i: (i, 0)),
            pl.BlockSpec((I, H), lambda i: (0, 0)),
            pl.BlockSpec((1, H), lambda i: (0, 0)),
            pl.BlockSpec((H, C), lambda i: (0, 0)),
            pl.BlockSpec((1, C), lambda i: (0, 0)),
        ],
        out_specs=pl.BlockSpec((1, 1, TB), lambda i: (i, 0, 0)),
        out_shape=jax.ShapeDtypeStruct((nb, 1, TB), jnp.int32),
    )(x, w1t, b1, w2t, b2)
    idx = idx3.reshape(N)

    wt = W_h2o.T
    bo = b_h2o.reshape(1, O)
    table = pl.pallas_call(
        _logsoftmax_body,
        grid=(C // ROWS_B,),
        in_specs=[
            pl.BlockSpec((ROWS_B, O), lambda i: (i, 0)),
            pl.BlockSpec((1, O), lambda i: (0, 0)),
        ],
        out_specs=pl.BlockSpec((ROWS_B, O), lambda i: (i, 0)),
        out_shape=jax.ShapeDtypeStruct((C, O), jnp.float32),
    )(wt, bo)

    info = plsc.get_sparse_core_info()
    nw = info.num_cores * info.num_subcores
    idx_3d = idx.reshape(nw, (N // nw) // CHUNK, CHUNK)
    out = _make_gather(N, O)(table, idx_3d)
    return out.reshape(B, S, O)


# speedup vs baseline: 1.1361x; 1.1361x over previous
import functools

import jax
import jax.numpy as jnp
from jax import lax
from jax.experimental import pallas as pl
from jax.experimental.pallas import tpu as pltpu
from jax.experimental.pallas import tpu_sc as plsc

TB = 256
ROWS_B = 512
CHUNK = 32
NBUF = 3
NCH_T = 4


def _argmax_body(x_ref, w1t_ref, b1_ref, w2t_ref, b2_ref, idx_ref):
    h = jnp.maximum(
        jnp.dot(x_ref[...], w1t_ref[...], preferred_element_type=jnp.float32)
        + b1_ref[...], 0.0)
    s = jnp.dot(h, w2t_ref[...], preferred_element_type=jnp.float32) + b2_ref[...]
    m = jnp.max(s, axis=-1, keepdims=True)
    col = lax.broadcasted_iota(jnp.int32, s.shape, 1)
    idx = jnp.min(jnp.where(s == m, col, jnp.int32(2**30)), axis=-1)
    idx_ref[0, 0, :] = idx


def _logsoftmax_body(wt_ref, b_ref, out_ref):
    z = wt_ref[...] + b_ref[...]
    m = jnp.max(z, axis=-1, keepdims=True)
    e = jnp.exp(z - m)
    lse = m + jnp.log(jnp.sum(e, axis=-1, keepdims=True))
    out_ref[...] = z - lse


def _make_gather(n_tok_chunk, d, chunk_off):
    info = plsc.get_sparse_core_info()
    nc, ns = info.num_cores, info.num_subcores
    nw = nc * ns
    b_per_w = n_tok_chunk // nw
    n_steps = b_per_w // CHUNK
    mesh = plsc.VectorSubcoreMesh(core_axis_name="c", subcore_axis_name="s")

    @functools.partial(
        pl.kernel,
        mesh=mesh,
        out_type=(),
        scratch_types=[
            pltpu.VMEM((n_steps, CHUNK), jnp.int32),
            pltpu.VMEM((NBUF, CHUNK, d), jnp.float32),
            pltpu.SemaphoreType.DMA,
            pltpu.SemaphoreType.DMA,
        ],
    )
    def gather_k(table_hbm, idx_hbm, out_hbm, idx_v, rows_v, sem_g, sem_o):
        wid = lax.axis_index("s") * nc + lax.axis_index("c")
        base = chunk_off + wid * b_per_w
        pltpu.sync_copy(idx_hbm.at[wid], idx_v)
        g = [None] * n_steps
        o = [None] * n_steps
        for i in range(n_steps):
            b = i % NBUF
            if i >= NBUF:
                o[i - NBUF].wait()
            g[i] = pltpu.async_copy(table_hbm.at[idx_v.at[i]], rows_v.at[b],
                                    sem_g)
            if i >= 1:
                g[i - 1].wait()
                o[i - 1] = pltpu.async_copy(
                    rows_v.at[(i - 1) % NBUF],
                    out_hbm.at[pl.ds(base + (i - 1) * CHUNK, CHUNK)], sem_o)
        g[n_steps - 1].wait()
        o[n_steps - 1] = pltpu.async_copy(
            rows_v.at[(n_steps - 1) % NBUF],
            out_hbm.at[pl.ds(base + (n_steps - 1) * CHUNK, CHUNK)], sem_o)
        for i in range(max(0, n_steps - NBUF), n_steps):
            o[i].wait()

    return gather_k


def kernel(input, hidden1, W_i2m, b_i2m, W_m2h, b_m2h, W_h2o, b_h2o):
    B, S, I = input.shape
    N = B * S
    H = W_i2m.shape[0]
    C = W_m2h.shape[0]
    O = W_h2o.shape[0]

    x = input.reshape(N, I)
    w1t = W_i2m.T
    w2t = W_m2h.T
    b1 = b_i2m.reshape(1, H)
    b2 = b_m2h.reshape(1, C)

    wt = W_h2o.T
    bo = b_h2o.reshape(1, O)
    table = pl.pallas_call(
        _logsoftmax_body,
        grid=(C // ROWS_B,),
        in_specs=[
            pl.BlockSpec((ROWS_B, O), lambda i: (i, 0)),
            pl.BlockSpec((1, O), lambda i: (0, 0)),
        ],
        out_specs=pl.BlockSpec((ROWS_B, O), lambda i: (i, 0)),
        out_shape=jax.ShapeDtypeStruct((C, O), jnp.float32),
    )(wt, bo)

    info = plsc.get_sparse_core_info()
    nw = info.num_cores * info.num_subcores

    tpc = N // NCH_T
    nb = tpc // TB
    out_ref = jax.new_ref(jnp.zeros((N, O), jnp.float32))
    for c in range(NCH_T):
        xc = lax.slice_in_dim(x, c * tpc, (c + 1) * tpc, axis=0)
        idx3 = pl.pallas_call(
            _argmax_body,
            grid=(nb,),
            in_specs=[
                pl.BlockSpec((TB, I), lambda i: (i, 0)),
                pl.BlockSpec((I, H), lambda i: (0, 0)),
                pl.BlockSpec((1, H), lambda i: (0, 0)),
                pl.BlockSpec((H, C), lambda i: (0, 0)),
                pl.BlockSpec((1, C), lambda i: (0, 0)),
            ],
            out_specs=pl.BlockSpec((1, 1, TB), lambda i: (i, 0, 0)),
            out_shape=jax.ShapeDtypeStruct((nb, 1, TB), jnp.int32),
        )(xc, w1t, b1, w2t, b2)
        idx_3d = idx3.reshape(nw, (tpc // nw) // CHUNK, CHUNK)
        _make_gather(tpc, O, c * tpc)(table, idx_3d, out_ref)

    return out_ref[...].reshape(B, S, O)


# speedup vs baseline: 1.2180x; 1.0722x over previous
import functools

import jax
import jax.numpy as jnp
from jax import lax
from jax.experimental import pallas as pl
from jax.experimental.pallas import tpu as pltpu
from jax.experimental.pallas import tpu_sc as plsc

TB = 128
ROWS_B = 512
CHUNK = 32
NBUF = 3
NCH_T = 4
LG = 128


def _argmax_body(x_ref, w1t_ref, b1_ref, w2ta_ref, idx_ref):
    h = jnp.maximum(
        jnp.dot(x_ref[...], w1t_ref[...], preferred_element_type=jnp.float32)
        + b1_ref[...], 0.0)
    ha = jnp.concatenate([h, jnp.ones((h.shape[0], 8), jnp.float32)], axis=1)
    s = jnp.dot(ha, w2ta_ref[...], preferred_element_type=jnp.float32)
    n_grp = s.shape[1] // LG
    val = s[:, 0:LG]
    gidx = jnp.zeros(val.shape, jnp.int32)
    for j in range(1, n_grp):
        v = s[:, j * LG:(j + 1) * LG]
        p = v > val
        val = jnp.where(p, v, val)
        gidx = jnp.where(p, jnp.int32(j), gidx)
    m = jnp.max(val, axis=-1, keepdims=True)
    lane = lax.broadcasted_iota(jnp.int32, val.shape, 1)
    lin = gidx * LG + lane
    idx = jnp.min(jnp.where(val == m, lin, jnp.int32(2**30)), axis=-1)
    idx_ref[0, 0, :] = idx


def _logsoftmax_body(wt_ref, b_ref, out_ref):
    z = wt_ref[...] + b_ref[...]
    m = jnp.max(z, axis=-1, keepdims=True)
    e = jnp.exp(z - m)
    lse = m + jnp.log(jnp.sum(e, axis=-1, keepdims=True))
    out_ref[...] = z - lse


def _make_gather(n_tok_chunk, n_tok_total, d, chunk_off, alloc_out):
    info = plsc.get_sparse_core_info()
    nc, ns = info.num_cores, info.num_subcores
    nw = nc * ns
    b_per_w = n_tok_chunk // nw
    n_steps = b_per_w // CHUNK
    mesh = plsc.VectorSubcoreMesh(core_axis_name="c", subcore_axis_name="s")

    @functools.partial(
        pl.kernel,
        mesh=mesh,
        out_type=(jax.ShapeDtypeStruct((n_tok_total, d), jnp.float32)
                  if alloc_out else ()),
        scratch_types=[
            pltpu.VMEM((n_steps, CHUNK), jnp.int32),
            pltpu.VMEM((NBUF, CHUNK, d), jnp.float32),
            pltpu.SemaphoreType.DMA,
            pltpu.SemaphoreType.DMA,
        ],
    )
    def gather_k(table_hbm, idx_hbm, out_hbm, idx_v, rows_v, sem_g, sem_o):
        wid = lax.axis_index("s") * nc + lax.axis_index("c")
        base = chunk_off + wid * b_per_w
        pltpu.sync_copy(idx_hbm.at[wid], idx_v)
        g = [None] * n_steps
        o = [None] * n_steps
        for i in range(n_steps):
            b = i % NBUF
            if i >= NBUF:
                o[i - NBUF].wait()
            g[i] = pltpu.async_copy(table_hbm.at[idx_v.at[i]], rows_v.at[b],
                                    sem_g)
            if i >= 1:
                g[i - 1].wait()
                o[i - 1] = pltpu.async_copy(
                    rows_v.at[(i - 1) % NBUF],
                    out_hbm.at[pl.ds(base + (i - 1) * CHUNK, CHUNK)], sem_o)
        g[n_steps - 1].wait()
        o[n_steps - 1] = pltpu.async_copy(
            rows_v.at[(n_steps - 1) % NBUF],
            out_hbm.at[pl.ds(base + (n_steps - 1) * CHUNK, CHUNK)], sem_o)
        for i in range(max(0, n_steps - NBUF), n_steps):
            o[i].wait()

    return gather_k


def kernel(input, hidden1, W_i2m, b_i2m, W_m2h, b_m2h, W_h2o, b_h2o):
    B, S, I = input.shape
    N = B * S
    H = W_i2m.shape[0]
    C = W_m2h.shape[0]
    O = W_h2o.shape[0]

    x = input.reshape(N, I)
    w1t = W_i2m.T
    b1 = b_i2m.reshape(1, H)
    w2ta = jnp.zeros((H + 8, C), jnp.float32)
    w2ta = w2ta.at[:H].set(W_m2h.T).at[H].set(b_m2h)

    wt = W_h2o.T
    bo = b_h2o.reshape(1, O)
    table = pl.pallas_call(
        _logsoftmax_body,
        grid=(C // ROWS_B,),
        in_specs=[
            pl.BlockSpec((ROWS_B, O), lambda i: (i, 0)),
            pl.BlockSpec((1, O), lambda i: (0, 0)),
        ],
        out_specs=pl.BlockSpec((ROWS_B, O), lambda i: (i, 0)),
        out_shape=jax.ShapeDtypeStruct((C, O), jnp.float32),
    )(wt, bo)

    info = plsc.get_sparse_core_info()
    nw = info.num_cores * info.num_subcores

    tpc = N // NCH_T
    nb = tpc // TB
    out_ref = None
    for c in range(NCH_T):
        idx3 = pl.pallas_call(
            _argmax_body,
            grid=(nb,),
            in_specs=[
                pl.BlockSpec((TB, I), lambda i, c=c, nb=nb: (i + c * nb, 0)),
                pl.BlockSpec((I, H), lambda i: (0, 0)),
                pl.BlockSpec((1, H), lambda i: (0, 0)),
                pl.BlockSpec((H + 8, C), lambda i: (0, 0)),
            ],
            out_specs=pl.BlockSpec((1, 1, TB), lambda i: (i, 0, 0)),
            out_shape=jax.ShapeDtypeStruct((nb, 1, TB), jnp.int32),
        )(x, w1t, b1, w2ta)
        idx_3d = idx3.reshape(nw, (tpc // nw) // CHUNK, CHUNK)
        if c == 0:
            out0 = _make_gather(tpc, N, O, 0, True)(table, idx_3d)
            out_ref = jax.new_ref(out0)
        else:
            _make_gather(tpc, N, O, c * tpc, False)(table, idx_3d, out_ref)

    return out_ref[...].reshape(B, S, O)


# speedup vs baseline: 1.2731x; 1.0452x over previous
import functools

import jax
import jax.numpy as jnp
from jax import lax
from jax.experimental import pallas as pl
from jax.experimental.pallas import tpu as pltpu
from jax.experimental.pallas import tpu_sc as plsc

TB = 128
ROWS_B = 512
CHUNK = 32
NBUF = 3
NCH_T = 4
LG = 128


def _argmax_body(x_ref, w1t_ref, b1_ref, w2ta_ref, idx_ref):
    h = jnp.maximum(
        jnp.dot(x_ref[...], w1t_ref[...], preferred_element_type=jnp.float32)
        + b1_ref[...], 0.0)
    ha = jnp.concatenate(
        [h.astype(jnp.bfloat16),
         jnp.ones((h.shape[0], 8), jnp.bfloat16)], axis=1)
    s = jnp.dot(ha, w2ta_ref[...], preferred_element_type=jnp.float32)
    n_grp = s.shape[1] // LG
    val = s[:, 0:LG]
    gidx = jnp.zeros(val.shape, jnp.int32)
    for j in range(1, n_grp):
        v = s[:, j * LG:(j + 1) * LG]
        p = v > val
        val = jnp.where(p, v, val)
        gidx = jnp.where(p, jnp.int32(j), gidx)
    m = jnp.max(val, axis=-1, keepdims=True)
    lane = lax.broadcasted_iota(jnp.int32, val.shape, 1)
    lin = gidx * LG + lane
    idx = jnp.min(jnp.where(val == m, lin, jnp.int32(2**30)), axis=-1)
    idx_ref[0, 0, :] = idx


def _logsoftmax_body(wt_ref, b_ref, out_ref):
    z = wt_ref[...] + b_ref[...]
    m = jnp.max(z, axis=-1, keepdims=True)
    e = jnp.exp(z - m)
    lse = m + jnp.log(jnp.sum(e, axis=-1, keepdims=True))
    out_ref[...] = z - lse


def _make_gather(n_tok_chunk, n_tok_total, d, chunk_off, alloc_out):
    info = plsc.get_sparse_core_info()
    nc, ns = info.num_cores, info.num_subcores
    nw = nc * ns
    b_per_w = n_tok_chunk // nw
    n_steps = b_per_w // CHUNK
    mesh = plsc.VectorSubcoreMesh(core_axis_name="c", subcore_axis_name="s")

    @functools.partial(
        pl.kernel,
        mesh=mesh,
        out_type=(jax.ShapeDtypeStruct((n_tok_total, d), jnp.float32)
                  if alloc_out else ()),
        scratch_types=[
            pltpu.VMEM((n_steps, CHUNK), jnp.int32),
            pltpu.VMEM((NBUF, CHUNK, d), jnp.float32),
            pltpu.SemaphoreType.DMA,
            pltpu.SemaphoreType.DMA,
        ],
    )
    def gather_k(table_hbm, idx_hbm, out_hbm, idx_v, rows_v, sem_g, sem_o):
        wid = lax.axis_index("s") * nc + lax.axis_index("c")
        base = chunk_off + wid * b_per_w
        pltpu.sync_copy(idx_hbm.at[wid], idx_v)
        g = [None] * n_steps
        o = [None] * n_steps
        for i in range(n_steps):
            b = i % NBUF
            if i >= NBUF:
                o[i - NBUF].wait()
            g[i] = pltpu.async_copy(table_hbm.at[idx_v.at[i]], rows_v.at[b],
                                    sem_g)
            if i >= 1:
                g[i - 1].wait()
                o[i - 1] = pltpu.async_copy(
                    rows_v.at[(i - 1) % NBUF],
                    out_hbm.at[pl.ds(base + (i - 1) * CHUNK, CHUNK)], sem_o)
        g[n_steps - 1].wait()
        o[n_steps - 1] = pltpu.async_copy(
            rows_v.at[(n_steps - 1) % NBUF],
            out_hbm.at[pl.ds(base + (n_steps - 1) * CHUNK, CHUNK)], sem_o)
        for i in range(max(0, n_steps - NBUF), n_steps):
            o[i].wait()

    return gather_k


def kernel(input, hidden1, W_i2m, b_i2m, W_m2h, b_m2h, W_h2o, b_h2o):
    B, S, I = input.shape
    N = B * S
    H = W_i2m.shape[0]
    C = W_m2h.shape[0]
    O = W_h2o.shape[0]

    x = input.reshape(N, I)
    w1t = W_i2m.T
    b1 = b_i2m.reshape(1, H)
    w2ta = jnp.zeros((H + 8, C), jnp.float32)
    w2ta = w2ta.at[:H].set(W_m2h.T).at[H].set(b_m2h)
    w2ta = w2ta.astype(jnp.bfloat16)

    wt = W_h2o.T
    bo = b_h2o.reshape(1, O)
    table = pl.pallas_call(
        _logsoftmax_body,
        grid=(C // ROWS_B,),
        in_specs=[
            pl.BlockSpec((ROWS_B, O), lambda i: (i, 0)),
            pl.BlockSpec((1, O), lambda i: (0, 0)),
        ],
        out_specs=pl.BlockSpec((ROWS_B, O), lambda i: (i, 0)),
        out_shape=jax.ShapeDtypeStruct((C, O), jnp.float32),
    )(wt, bo)

    info = plsc.get_sparse_core_info()
    nw = info.num_cores * info.num_subcores

    tpc = N // NCH_T
    nb = tpc // TB
    out_ref = None
    for c in range(NCH_T):
        idx3 = pl.pallas_call(
            _argmax_body,
            grid=(nb,),
            in_specs=[
                pl.BlockSpec((TB, I), lambda i, c=c, nb=nb: (i + c * nb, 0)),
                pl.BlockSpec((I, H), lambda i: (0, 0)),
                pl.BlockSpec((1, H), lambda i: (0, 0)),
                pl.BlockSpec((H + 8, C), lambda i: (0, 0)),
            ],
            out_specs=pl.BlockSpec((1, 1, TB), lambda i: (i, 0, 0)),
            out_shape=jax.ShapeDtypeStruct((nb, 1, TB), jnp.int32),
        )(x, w1t, b1, w2ta)
        idx_3d = idx3.reshape(nw, (tpc // nw) // CHUNK, CHUNK)
        if c == 0:
            out0 = _make_gather(tpc, N, O, 0, True)(table, idx_3d)
            out_ref = jax.new_ref(out0)
        else:
            _make_gather(tpc, N, O, c * tpc, False)(table, idx_3d, out_ref)

    return out_ref[...].reshape(B, S, O)


# speedup vs baseline: 1.5297x; 1.2015x over previous
import functools

import jax
import jax.numpy as jnp
from jax import lax
from jax.experimental import pallas as pl
from jax.experimental.pallas import tpu as pltpu
from jax.experimental.pallas import tpu_sc as plsc

TB = 256
ROWS_B = 512
CHUNK = 32
NBUF = 3
NCH_T = 4
LG = 128


def _argmax_body(x_ref, w1t_ref, b1_ref, w2ta_ref, idx_ref):
    h = jnp.maximum(
        jnp.dot(x_ref[...], w1t_ref[...], preferred_element_type=jnp.float32)
        + b1_ref[...], 0.0)
    ha = jnp.concatenate(
        [h.astype(jnp.bfloat16),
         jnp.ones((h.shape[0], 8), jnp.bfloat16)], axis=1)
    s = jnp.dot(ha, w2ta_ref[...], preferred_element_type=jnp.float32)
    n_grp = s.shape[1] // LG
    val = s[:, 0:LG]
    gidx = jnp.zeros(val.shape, jnp.int32)
    for j in range(1, n_grp):
        v = s[:, j * LG:(j + 1) * LG]
        p = v > val
        val = jnp.where(p, v, val)
        gidx = jnp.where(p, jnp.int32(j), gidx)
    m = jnp.max(val, axis=-1, keepdims=True)
    lane = lax.broadcasted_iota(jnp.int32, val.shape, 1)
    lin = gidx * LG + lane
    idx = jnp.min(jnp.where(val == m, lin, jnp.int32(2**30)), axis=-1)
    idx_ref[0, 0, :] = idx


def _logsoftmax_body(wt_ref, b_ref, out_ref):
    z = wt_ref[...] + b_ref[...]
    m = jnp.max(z, axis=-1, keepdims=True)
    e = jnp.exp(z - m)
    lse = m + jnp.log(jnp.sum(e, axis=-1, keepdims=True))
    out_ref[...] = z - lse


def _make_gather(n_tok_chunk, n_tok_total, d, chunk_off, alloc_out):
    info = plsc.get_sparse_core_info()
    nc, ns = info.num_cores, info.num_subcores
    nw = nc * ns
    b_per_w = n_tok_chunk // nw
    n_steps = b_per_w // CHUNK
    mesh = plsc.VectorSubcoreMesh(core_axis_name="c", subcore_axis_name="s")

    @functools.partial(
        pl.kernel,
        mesh=mesh,
        out_type=(jax.ShapeDtypeStruct((n_tok_total, d), jnp.float32)
                  if alloc_out else ()),
        scratch_types=[
            pltpu.VMEM((n_steps, CHUNK), jnp.int32),
            pltpu.VMEM((NBUF, CHUNK, d), jnp.float32),
            pltpu.SemaphoreType.DMA,
            pltpu.SemaphoreType.DMA,
        ],
    )
    def gather_k(table_hbm, idx_hbm, out_hbm, idx_v, rows_v, sem_g, sem_o):
        wid = lax.axis_index("s") * nc + lax.axis_index("c")
        base = chunk_off + wid * b_per_w
        pltpu.sync_copy(idx_hbm.at[wid], idx_v)
        g = [None] * n_steps
        o = [None] * n_steps
        for i in range(n_steps):
            b = i % NBUF
            if i >= NBUF:
                o[i - NBUF].wait()
            g[i] = pltpu.async_copy(table_hbm.at[idx_v.at[i]], rows_v.at[b],
                                    sem_g)
            if i >= 1:
                g[i - 1].wait()
                o[i - 1] = pltpu.async_copy(
                    rows_v.at[(i - 1) % NBUF],
                    out_hbm.at[pl.ds(base + (i - 1) * CHUNK, CHUNK)], sem_o)
        g[n_steps - 1].wait()
        o[n_steps - 1] = pltpu.async_copy(
            rows_v.at[(n_steps - 1) % NBUF],
            out_hbm.at[pl.ds(base + (n_steps - 1) * CHUNK, CHUNK)], sem_o)
        for i in range(max(0, n_steps - NBUF), n_steps):
            o[i].wait()

    return gather_k


def kernel(input, hidden1, W_i2m, b_i2m, W_m2h, b_m2h, W_h2o, b_h2o):
    B, S, I = input.shape
    N = B * S
    H = W_i2m.shape[0]
    C = W_m2h.shape[0]
    O = W_h2o.shape[0]

    x = input.reshape(N, I)
    w1t = W_i2m.T
    b1 = b_i2m.reshape(1, H)
    w2ta = jnp.zeros((H + 8, C), jnp.float32)
    w2ta = w2ta.at[:H].set(W_m2h.T).at[H].set(b_m2h)
    w2ta = w2ta.astype(jnp.bfloat16)

    wt = W_h2o.T
    bo = b_h2o.reshape(1, O)
    table = pl.pallas_call(
        _logsoftmax_body,
        grid=(C // ROWS_B,),
        in_specs=[
            pl.BlockSpec((ROWS_B, O), lambda i: (i, 0)),
            pl.BlockSpec((1, O), lambda i: (0, 0)),
        ],
        out_specs=pl.BlockSpec((ROWS_B, O), lambda i: (i, 0)),
        out_shape=jax.ShapeDtypeStruct((C, O), jnp.float32),
    )(wt, bo)

    info = plsc.get_sparse_core_info()
    nw = info.num_cores * info.num_subcores

    tpc = N // NCH_T
    nb = tpc // TB
    out_ref = None
    for c in range(NCH_T):
        idx3 = pl.pallas_call(
            _argmax_body,
            grid=(nb,),
            in_specs=[
                pl.BlockSpec((TB, I), lambda i, c=c, nb=nb: (i + c * nb, 0)),
                pl.BlockSpec((I, H), lambda i: (0, 0)),
                pl.BlockSpec((1, H), lambda i: (0, 0)),
                pl.BlockSpec((H + 8, C), lambda i: (0, 0)),
            ],
            out_specs=pl.BlockSpec((1, 1, TB), lambda i: (i, 0, 0)),
            out_shape=jax.ShapeDtypeStruct((nb, 1, TB), jnp.int32),
        )(x, w1t, b1, w2ta)
        idx_3d = idx3.reshape(nw, (tpc // nw) // CHUNK, CHUNK)
        if c == 0:
            out0 = _make_gather(tpc, N, O, 0, True)(table, idx_3d)
            out_ref = jax.new_ref(out0)
        else:
            _make_gather(tpc, N, O, c * tpc, False)(table, idx_3d, out_ref)

    return out_ref[...].reshape(B, S, O)


# speedup vs baseline: 1.6302x; 1.0657x over previous
import functools

import jax
import jax.numpy as jnp
from jax import lax
from jax.experimental import pallas as pl
from jax.experimental.pallas import tpu as pltpu
from jax.experimental.pallas import tpu_sc as plsc

TB = 512
ROWS_B = 512
CHUNK = 32
NBUF = 3
NCH_T = 4
LG = 128


def _argmax_body(x_ref, w1t_ref, b1_ref, w2ta_ref, idx_ref):
    h = jnp.maximum(
        jnp.dot(x_ref[...], w1t_ref[...], preferred_element_type=jnp.float32)
        + b1_ref[...], 0.0)
    ha = jnp.concatenate(
        [h.astype(jnp.bfloat16),
         jnp.ones((h.shape[0], 8), jnp.bfloat16)], axis=1)
    s = jnp.dot(ha, w2ta_ref[...], preferred_element_type=jnp.float32)
    n_grp = s.shape[1] // LG
    val = s[:, 0:LG]
    gidx = jnp.zeros(val.shape, jnp.int32)
    for j in range(1, n_grp):
        v = s[:, j * LG:(j + 1) * LG]
        p = v > val
        val = jnp.where(p, v, val)
        gidx = jnp.where(p, jnp.int32(j), gidx)
    m = jnp.max(val, axis=-1, keepdims=True)
    lane = lax.broadcasted_iota(jnp.int32, val.shape, 1)
    lin = gidx * LG + lane
    idx = jnp.min(jnp.where(val == m, lin, jnp.int32(2**30)), axis=-1)
    idx_ref[0, 0, :] = idx


def _logsoftmax_body(wt_ref, b_ref, out_ref):
    z = wt_ref[...] + b_ref[...]
    m = jnp.max(z, axis=-1, keepdims=True)
    e = jnp.exp(z - m)
    lse = m + jnp.log(jnp.sum(e, axis=-1, keepdims=True))
    out_ref[...] = z - lse


def _make_gather(n_tok_chunk, n_tok_total, d, chunk_off, alloc_out):
    info = plsc.get_sparse_core_info()
    nc, ns = info.num_cores, info.num_subcores
    nw = nc * ns
    b_per_w = n_tok_chunk // nw
    n_steps = b_per_w // CHUNK
    mesh = plsc.VectorSubcoreMesh(core_axis_name="c", subcore_axis_name="s")

    @functools.partial(
        pl.kernel,
        mesh=mesh,
        out_type=(jax.ShapeDtypeStruct((n_tok_total, d), jnp.float32)
                  if alloc_out else ()),
        scratch_types=[
            pltpu.VMEM((n_steps, CHUNK), jnp.int32),
            pltpu.VMEM((NBUF, CHUNK, d), jnp.float32),
            pltpu.SemaphoreType.DMA,
            pltpu.SemaphoreType.DMA,
        ],
    )
    def gather_k(table_hbm, idx_hbm, out_hbm, idx_v, rows_v, sem_g, sem_o):
        wid = lax.axis_index("s") * nc + lax.axis_index("c")
        base = chunk_off + wid * b_per_w
        pltpu.sync_copy(idx_hbm.at[wid], idx_v)
        g = [None] * n_steps
        o = [None] * n_steps
        for i in range(n_steps):
            b = i % NBUF
            if i >= NBUF:
                o[i - NBUF].wait()
            g[i] = pltpu.async_copy(table_hbm.at[idx_v.at[i]], rows_v.at[b],
                                    sem_g)
            if i >= 1:
                g[i - 1].wait()
                o[i - 1] = pltpu.async_copy(
                    rows_v.at[(i - 1) % NBUF],
                    out_hbm.at[pl.ds(base + (i - 1) * CHUNK, CHUNK)], sem_o)
        g[n_steps - 1].wait()
        o[n_steps - 1] = pltpu.async_copy(
            rows_v.at[(n_steps - 1) % NBUF],
            out_hbm.at[pl.ds(base + (n_steps - 1) * CHUNK, CHUNK)], sem_o)
        for i in range(max(0, n_steps - NBUF), n_steps):
            o[i].wait()

    return gather_k


def kernel(input, hidden1, W_i2m, b_i2m, W_m2h, b_m2h, W_h2o, b_h2o):
    B, S, I = input.shape
    N = B * S
    H = W_i2m.shape[0]
    C = W_m2h.shape[0]
    O = W_h2o.shape[0]

    x = input.reshape(N, I)
    w1t = W_i2m.T
    b1 = b_i2m.reshape(1, H)
    w2ta = jnp.zeros((H + 8, C), jnp.float32)
    w2ta = w2ta.at[:H].set(W_m2h.T).at[H].set(b_m2h)
    w2ta = w2ta.astype(jnp.bfloat16)

    wt = W_h2o.T
    bo = b_h2o.reshape(1, O)
    table = pl.pallas_call(
        _logsoftmax_body,
        grid=(C // ROWS_B,),
        in_specs=[
            pl.BlockSpec((ROWS_B, O), lambda i: (i, 0)),
            pl.BlockSpec((1, O), lambda i: (0, 0)),
        ],
        out_specs=pl.BlockSpec((ROWS_B, O), lambda i: (i, 0)),
        out_shape=jax.ShapeDtypeStruct((C, O), jnp.float32),
    )(wt, bo)

    info = plsc.get_sparse_core_info()
    nw = info.num_cores * info.num_subcores

    tpc = N // NCH_T
    nb = tpc // TB
    out_ref = None
    for c in range(NCH_T):
        idx3 = pl.pallas_call(
            _argmax_body,
            grid=(nb,),
            in_specs=[
                pl.BlockSpec((TB, I), lambda i, c=c, nb=nb: (i + c * nb, 0)),
                pl.BlockSpec((I, H), lambda i: (0, 0)),
                pl.BlockSpec((1, H), lambda i: (0, 0)),
                pl.BlockSpec((H + 8, C), lambda i: (0, 0)),
            ],
            out_specs=pl.BlockSpec((1, 1, TB), lambda i: (i, 0, 0)),
            out_shape=jax.ShapeDtypeStruct((nb, 1, TB), jnp.int32),
        )(x, w1t, b1, w2ta)
        idx_3d = idx3.reshape(nw, (tpc // nw) // CHUNK, CHUNK)
        if c == 0:
            out0 = _make_gather(tpc, N, O, 0, True)(table, idx_3d)
            out_ref = jax.new_ref(out0)
        else:
            _make_gather(tpc, N, O, c * tpc, False)(table, idx_3d, out_ref)

    return out_ref[...].reshape(B, S, O)


# speedup vs baseline: 1.6332x; 1.0019x over previous
import functools

import jax
import jax.numpy as jnp
from jax import lax
from jax.experimental import pallas as pl
from jax.experimental.pallas import tpu as pltpu
from jax.experimental.pallas import tpu_sc as plsc

TB = 1024
ROWS_B = 512
CHUNK = 32
NBUF = 3
NCH_T = 4
LG = 128


def _argmax_body(x_ref, w1t_ref, b1_ref, w2ta_ref, idx_ref):
    h = jnp.maximum(
        jnp.dot(x_ref[...], w1t_ref[...], preferred_element_type=jnp.float32)
        + b1_ref[...], 0.0)
    ha = jnp.concatenate(
        [h.astype(jnp.bfloat16),
         jnp.ones((h.shape[0], 8), jnp.bfloat16)], axis=1)
    s = jnp.dot(ha, w2ta_ref[...], preferred_element_type=jnp.float32)
    n_grp = s.shape[1] // LG
    val = s[:, 0:LG]
    gidx = jnp.zeros(val.shape, jnp.int32)
    for j in range(1, n_grp):
        v = s[:, j * LG:(j + 1) * LG]
        p = v > val
        val = jnp.where(p, v, val)
        gidx = jnp.where(p, jnp.int32(j), gidx)
    m = jnp.max(val, axis=-1, keepdims=True)
    lane = lax.broadcasted_iota(jnp.int32, val.shape, 1)
    lin = gidx * LG + lane
    idx = jnp.min(jnp.where(val == m, lin, jnp.int32(2**30)), axis=-1)
    idx_ref[0, 0, :] = idx


def _logsoftmax_body(wt_ref, b_ref, out_ref):
    z = wt_ref[...] + b_ref[...]
    m = jnp.max(z, axis=-1, keepdims=True)
    e = jnp.exp(z - m)
    lse = m + jnp.log(jnp.sum(e, axis=-1, keepdims=True))
    out_ref[...] = z - lse


def _make_gather(n_tok_chunk, n_tok_total, d, chunk_off, alloc_out):
    info = plsc.get_sparse_core_info()
    nc, ns = info.num_cores, info.num_subcores
    nw = nc * ns
    b_per_w = n_tok_chunk // nw
    n_steps = b_per_w // CHUNK
    mesh = plsc.VectorSubcoreMesh(core_axis_name="c", subcore_axis_name="s")

    @functools.partial(
        pl.kernel,
        mesh=mesh,
        out_type=(jax.ShapeDtypeStruct((n_tok_total, d), jnp.float32)
                  if alloc_out else ()),
        scratch_types=[
            pltpu.VMEM((n_steps, CHUNK), jnp.int32),
            pltpu.VMEM((NBUF, CHUNK, d), jnp.float32),
            pltpu.SemaphoreType.DMA,
            pltpu.SemaphoreType.DMA,
        ],
    )
    def gather_k(table_hbm, idx_hbm, out_hbm, idx_v, rows_v, sem_g, sem_o):
        wid = lax.axis_index("s") * nc + lax.axis_index("c")
        base = chunk_off + wid * b_per_w
        pltpu.sync_copy(idx_hbm.at[wid], idx_v)
        g = [None] * n_steps
        o = [None] * n_steps
        for i in range(n_steps):
            b = i % NBUF
            if i >= NBUF:
                o[i - NBUF].wait()
            g[i] = pltpu.async_copy(table_hbm.at[idx_v.at[i]], rows_v.at[b],
                                    sem_g)
            if i >= 1:
                g[i - 1].wait()
                o[i - 1] = pltpu.async_copy(
                    rows_v.at[(i - 1) % NBUF],
                    out_hbm.at[pl.ds(base + (i - 1) * CHUNK, CHUNK)], sem_o)
        g[n_steps - 1].wait()
        o[n_steps - 1] = pltpu.async_copy(
            rows_v.at[(n_steps - 1) % NBUF],
            out_hbm.at[pl.ds(base + (n_steps - 1) * CHUNK, CHUNK)], sem_o)
        for i in range(max(0, n_steps - NBUF), n_steps):
            o[i].wait()

    return gather_k


def kernel(input, hidden1, W_i2m, b_i2m, W_m2h, b_m2h, W_h2o, b_h2o):
    B, S, I = input.shape
    N = B * S
    H = W_i2m.shape[0]
    C = W_m2h.shape[0]
    O = W_h2o.shape[0]

    x = input.reshape(N, I)
    w1t = W_i2m.T
    b1 = b_i2m.reshape(1, H)
    w2ta = jnp.zeros((H + 8, C), jnp.float32)
    w2ta = w2ta.at[:H].set(W_m2h.T).at[H].set(b_m2h)
    w2ta = w2ta.astype(jnp.bfloat16)

    wt = W_h2o.T
    bo = b_h2o.reshape(1, O)
    table = pl.pallas_call(
        _logsoftmax_body,
        grid=(C // ROWS_B,),
        in_specs=[
            pl.BlockSpec((ROWS_B, O), lambda i: (i, 0)),
            pl.BlockSpec((1, O), lambda i: (0, 0)),
        ],
        out_specs=pl.BlockSpec((ROWS_B, O), lambda i: (i, 0)),
        out_shape=jax.ShapeDtypeStruct((C, O), jnp.float32),
    )(wt, bo)

    info = plsc.get_sparse_core_info()
    nw = info.num_cores * info.num_subcores

    tpc = N // NCH_T
    nb = tpc // TB
    out_ref = None
    for c in range(NCH_T):
        idx3 = pl.pallas_call(
            _argmax_body,
            grid=(nb,),
            in_specs=[
                pl.BlockSpec((TB, I), lambda i, c=c, nb=nb: (i + c * nb, 0)),
                pl.BlockSpec((I, H), lambda i: (0, 0)),
                pl.BlockSpec((1, H), lambda i: (0, 0)),
                pl.BlockSpec((H + 8, C), lambda i: (0, 0)),
            ],
            out_specs=pl.BlockSpec((1, 1, TB), lambda i: (i, 0, 0)),
            out_shape=jax.ShapeDtypeStruct((nb, 1, TB), jnp.int32),
        )(x, w1t, b1, w2ta)
        idx_3d = idx3.reshape(nw, (tpc // nw) // CHUNK, CHUNK)
        if c == 0:
            out0 = _make_gather(tpc, N, O, 0, True)(table, idx_3d)
            out_ref = jax.new_ref(out0)
        else:
            _make_gather(tpc, N, O, c * tpc, False)(table, idx_3d, out_ref)

    return out_ref[...].reshape(B, S, O)
